# baseline (speedup 1.0000x reference)
import jax
import jax.numpy as jnp
from jax.experimental import pallas as pl

_CT_T = (((1,), (1,)), ((), ()))
_CT_00 = (((0,), (0,)), ((), ()))
_CT_N = (((1,), (0,)), ((), ()))


def _netmon_kernel(x_ref, adj_ref, nam_ref, w0_ref, b0_ref, w1_ref, b1_ref,
                   w2_ref, b2_ref, out_ref):
    f32 = jnp.float32
    xb = x_ref[0]
    adj = adj_ref[0]
    nam = nam_ref[0]

    h = jax.lax.dot_general(xb, w0_ref[...], _CT_T, preferred_element_type=f32)
    h = jnp.maximum(h + b0_ref[...], 0.0)
    h = jax.lax.dot_general(h, w1_ref[...], _CT_T, preferred_element_type=f32)
    h = jnp.maximum(h + b1_ref[...], 0.0)
    h = jax.lax.dot_general(h, w2_ref[...], _CT_T, preferred_element_type=f32)
    h = jnp.maximum(h + b2_ref[...], 0.0)

    v = jax.lax.dot_general(nam, adj, _CT_00, preferred_element_type=f32)
    v = jax.lax.dot_general(v, adj, _CT_N, preferred_element_type=f32)
    v = jax.lax.dot_general(v, adj, _CT_N, preferred_element_type=f32)
    out_ref[0] = jax.lax.dot_general(v, h, _CT_N, preferred_element_type=f32)


def kernel(x, node_adjacency, node_agent_matrix, W0, b0, W1, b1, W2, b2):
    B, N, IN_F = x.shape
    A = node_agent_matrix.shape[-1]
    HID = W2.shape[0]

    grid = (B,)
    out = pl.pallas_call(
        _netmon_kernel,
        grid=grid,
        in_specs=[
            pl.BlockSpec((1, N, IN_F), lambda b: (b, 0, 0)),
            pl.BlockSpec((1, N, N), lambda b: (b, 0, 0)),
            pl.BlockSpec((1, N, A), lambda b: (b, 0, 0)),
            pl.BlockSpec(W0.shape, lambda b: (0, 0)),
            pl.BlockSpec((1, b0.shape[0]), lambda b: (0, 0)),
            pl.BlockSpec(W1.shape, lambda b: (0, 0)),
            pl.BlockSpec((1, b1.shape[0]), lambda b: (0, 0)),
            pl.BlockSpec(W2.shape, lambda b: (0, 0)),
            pl.BlockSpec((1, b2.shape[0]), lambda b: (0, 0)),
        ],
        out_specs=pl.BlockSpec((1, A, HID), lambda b: (b, 0, 0)),
        out_shape=jax.ShapeDtypeStruct((B, A, HID), jnp.float32),
    )(x, node_adjacency, node_agent_matrix,
      W0, b0.reshape(1, -1), W1, b1.reshape(1, -1), W2, b2.reshape(1, -1))
    return out


# speedup vs baseline: 1.6512x; 1.6512x over previous
import jax
import jax.numpy as jnp
from jax.experimental import pallas as pl

_CT_T = (((1,), (1,)), ((), ()))
_CT_00 = (((0,), (0,)), ((), ()))
_CT_N = (((1,), (0,)), ((), ()))


_G = 8


def _netmon_kernel(x_ref, adj_ref, nam_ref, w0_ref, b0_ref, w1_ref, b1_ref,
                   w2_ref, b2_ref, out_ref):
    f32 = jnp.float32
    G, N, IN_F = x_ref.shape
    xb = x_ref[...].reshape(G * N, IN_F)

    h = jax.lax.dot_general(xb, w0_ref[...], _CT_T, preferred_element_type=f32)
    h = jnp.maximum(h + b0_ref[...], 0.0)
    h = jax.lax.dot_general(h, w1_ref[...], _CT_T, preferred_element_type=f32)
    h = jnp.maximum(h + b1_ref[...], 0.0)
    h = jax.lax.dot_general(h, w2_ref[...], _CT_T, preferred_element_type=f32)
    h = jnp.maximum(h + b2_ref[...], 0.0)

    for g in range(G):
        adj = adj_ref[g]
        nam = nam_ref[g]
        hg = h[g * N:(g + 1) * N]
        v = jax.lax.dot_general(nam, adj, _CT_00, preferred_element_type=f32)
        v = jax.lax.dot_general(v, adj, _CT_N, preferred_element_type=f32)
        v = jax.lax.dot_general(v, adj, _CT_N, preferred_element_type=f32)
        out_ref[g] = jax.lax.dot_general(v, hg, _CT_N, preferred_element_type=f32)


def kernel(x, node_adjacency, node_agent_matrix, W0, b0, W1, b1, W2, b2):
    B, N, IN_F = x.shape
    A = node_agent_matrix.shape[-1]
    HID = W2.shape[0]

    grid = (B // _G,)
    out = pl.pallas_call(
        _netmon_kernel,
        grid=grid,
        in_specs=[
            pl.BlockSpec((_G, N, IN_F), lambda b: (b, 0, 0)),
            pl.BlockSpec((_G, N, N), lambda b: (b, 0, 0)),
            pl.BlockSpec((_G, N, A), lambda b: (b, 0, 0)),
            pl.BlockSpec(W0.shape, lambda b: (0, 0)),
            pl.BlockSpec((1, b0.shape[0]), lambda b: (0, 0)),
            pl.BlockSpec(W1.shape, lambda b: (0, 0)),
            pl.BlockSpec((1, b1.shape[0]), lambda b: (0, 0)),
            pl.BlockSpec(W2.shape, lambda b: (0, 0)),
            pl.BlockSpec((1, b2.shape[0]), lambda b: (0, 0)),
        ],
        out_specs=pl.BlockSpec((_G, A, HID), lambda b: (b, 0, 0)),
        out_shape=jax.ShapeDtypeStruct((B, A, HID), jnp.float32),
    )(x, node_adjacency, node_agent_matrix,
      W0, b0.reshape(1, -1), W1, b1.reshape(1, -1), W2, b2.reshape(1, -1))
    return out
